# baseline (speedup 1.0000x reference)
import functools

import jax
import jax.numpy as jnp
from jax import lax
from jax.experimental import pallas as pl
from jax.experimental.pallas import tpu as pltpu
from jax.experimental.pallas import tpu_sc as plsc

N_NODES = 64
N_GROUPS = 100000
BATCH = 16384

NC = 2
NS = 16
NW = NC * NS
LANES = 16

BPW = BATCH // NW
GCH = 128
NG = BPW // GCH
WROW = 128

BM_W = 16384
BM_T = 16384


def _tp_w_kernel(wt_ref, out_ref):
    x = wt_ref[...]
    xt = x.T
    pad = jnp.zeros((BM_W, WROW - N_NODES), jnp.float32)
    out_ref[...] = jnp.concatenate([xt, pad], axis=1)


def _tp_t_kernel(tt_ref, out_ref):
    out_ref[...] = tt_ref[...].T


def _sc_dot_kernel(t_hbm, c_hbm, w_hbm, out_hbm, idx_v, rows_v, t_v, buf_v,
                   out_v, gsem, tsem):
    wid = lax.axis_index("s") * NC + lax.axis_index("c")
    base = pl.multiple_of(wid * BPW, BPW)
    tbase = pl.multiple_of(wid * (BPW // 2), BPW // 2)

    pltpu.sync_copy(c_hbm.at[wid], idx_v)

    t_cp = pltpu.async_copy(t_hbm.at[pl.ds(base, BPW)], t_v, tsem)

    def fire(g):
        return pltpu.async_copy(w_hbm.at[idx_v.at[g]], rows_v.at[g % 3], gsem)

    gathers = [fire(0), fire(1), fire(2)]
    t_cp.wait()

    lane_iota = lax.iota(jnp.int32, LANES)
    nchunk = N_NODES // LANES

    for g in range(NG):
        gathers[g].wait()
        rb = rows_v.at[g % 3]

        @pl.loop(0, GCH, step=16)
        def _group(r0):
            row0 = g * GCH + r0
            for r in range(16):
                lrow = r0 + r
                grow = row0 + r
                acc = (rb[lrow, pl.ds(0, LANES)]
                       * t_v[grow, pl.ds(0, LANES)])
                for k in range(1, nchunk):
                    acc = acc + (rb[lrow, pl.ds(k * LANES, LANES)]
                                 * t_v[grow, pl.ds(k * LANES, LANES)])
                plsc.store_scatter(
                    buf_v, [lane_iota, jnp.full((LANES,), r, jnp.int32)], acc)
            tot = buf_v[0, :]
            for l in range(1, 16):
                tot = tot + buf_v[l, :]
            out_v[pl.ds(row0, 16)] = tot

        if g + 3 < NG:
            gathers.append(fire(g + 3))

    pltpu.sync_copy(out_v, out_hbm.at[pl.ds(base, BPW)])


@jax.jit
def kernel(t, c, W):
    c2 = c.reshape(NW, NG, GCH).astype(jnp.int32)

    tc_params = pltpu.CompilerParams(dimension_semantics=("parallel",))
    wp = pl.pallas_call(
        _tp_w_kernel,
        out_shape=jax.ShapeDtypeStruct((N_GROUPS, WROW), jnp.float32),
        grid=((N_GROUPS + BM_W - 1) // BM_W,),
        in_specs=[pl.BlockSpec((N_NODES, BM_W), lambda i: (0, i))],
        out_specs=pl.BlockSpec((BM_W, WROW), lambda i: (i, 0)),
        compiler_params=tc_params,
    )(W.T)

    t2 = pl.pallas_call(
        _tp_t_kernel,
        out_shape=jax.ShapeDtypeStruct((BATCH, N_NODES), jnp.float32),
        grid=(BATCH // BM_T,),
        in_specs=[pl.BlockSpec((N_NODES, BM_T), lambda i: (0, i))],
        out_specs=pl.BlockSpec((BM_T, N_NODES), lambda i: (i, 0)),
        compiler_params=tc_params,
    )(t.T)

    mesh = plsc.VectorSubcoreMesh(core_axis_name="c", subcore_axis_name="s")
    cp = pltpu.CompilerParams(needs_layout_passes=False)
    run = functools.partial(
        pl.kernel,
        mesh=mesh,
        compiler_params=cp,
        out_type=jax.ShapeDtypeStruct((BATCH,), jnp.float32),
        scratch_types=[
            pltpu.VMEM((NG, GCH), jnp.int32),
            pltpu.VMEM((3, GCH, WROW), jnp.float32),
            pltpu.VMEM((BPW, N_NODES), jnp.float32),
            pltpu.VMEM((LANES, LANES), jnp.float32),
            pltpu.VMEM((BPW,), jnp.float32),
            pltpu.SemaphoreType.DMA,
            pltpu.SemaphoreType.DMA,
        ],
    )(_sc_dot_kernel)
    return run(t2, c2, wp)


# speedup vs baseline: 1.0373x; 1.0373x over previous
import functools

import jax
import jax.numpy as jnp
from jax import lax
from jax.experimental import pallas as pl
from jax.experimental.pallas import tpu as pltpu
from jax.experimental.pallas import tpu_sc as plsc

N_NODES = 64
N_GROUPS = 100000
BATCH = 16384

NC = 2
NS = 16
NW = NC * NS
LANES = 16

BPW = BATCH // NW
GCH = 128
NG = BPW // GCH
WROW = 128

BM_W = 16384
BM_T = 8192


def _tp_w_kernel(wt_ref, out_ref):
    x = wt_ref[...]
    xt = x.T
    pad = jnp.zeros((BM_W, WROW - N_NODES), jnp.float32)
    out_ref[...] = jnp.concatenate([xt, pad], axis=1)


def _tp_t_kernel(tt_ref, out_ref):
    out_ref[...] = tt_ref[...].T


def _sc_dot_kernel(t_hbm, c_hbm, w_hbm, out_hbm, idx_v, rows_v, t_v, buf_v,
                   out_v, gsem, tsem):
    wid = lax.axis_index("s") * NC + lax.axis_index("c")
    base = pl.multiple_of(wid * BPW, BPW)
    tbase = pl.multiple_of(wid * (BPW // 2), BPW // 2)

    pltpu.sync_copy(c_hbm.at[wid], idx_v)

    t_cp = pltpu.async_copy(t_hbm.at[pl.ds(base, BPW)], t_v, tsem)

    def fire(g):
        return pltpu.async_copy(w_hbm.at[idx_v.at[g]], rows_v.at[g % 2], gsem)

    gathers = [fire(0), fire(1)]
    t_cp.wait()

    lane_iota = lax.iota(jnp.int32, LANES)
    nchunk = N_NODES // LANES

    for g in range(NG):
        gathers[g].wait()
        rb = rows_v.at[g % 2]

        @pl.loop(0, GCH, step=16)
        def _group(r0):
            row0 = g * GCH + r0
            for r in range(16):
                lrow = r0 + r
                grow = row0 + r
                acc = (rb[lrow, pl.ds(0, LANES)]
                       * t_v[grow, pl.ds(0, LANES)])
                for k in range(1, nchunk):
                    acc = acc + (rb[lrow, pl.ds(k * LANES, LANES)]
                                 * t_v[grow, pl.ds(k * LANES, LANES)])
                plsc.store_scatter(
                    buf_v, [lane_iota, jnp.full((LANES,), r, jnp.int32)], acc)
            tot = buf_v[0, :]
            for l in range(1, 16):
                tot = tot + buf_v[l, :]
            out_v[pl.ds(row0, 16)] = tot

        if g + 2 < NG:
            gathers.append(fire(g + 2))

    pltpu.sync_copy(out_v, out_hbm.at[pl.ds(base, BPW)])


@jax.jit
def kernel(t, c, W):
    c2 = c.reshape(NW, NG, GCH).astype(jnp.int32)

    tc_params = pltpu.CompilerParams(dimension_semantics=("parallel",))
    wp = pl.pallas_call(
        _tp_w_kernel,
        out_shape=jax.ShapeDtypeStruct((N_GROUPS, WROW), jnp.float32),
        grid=((N_GROUPS + BM_W - 1) // BM_W,),
        in_specs=[pl.BlockSpec((N_NODES, BM_W), lambda i: (0, i))],
        out_specs=pl.BlockSpec((BM_W, WROW), lambda i: (i, 0)),
        compiler_params=tc_params,
    )(W.T)

    t2 = pl.pallas_call(
        _tp_t_kernel,
        out_shape=jax.ShapeDtypeStruct((BATCH, N_NODES), jnp.float32),
        grid=(BATCH // BM_T,),
        in_specs=[pl.BlockSpec((N_NODES, BM_T), lambda i: (0, i))],
        out_specs=pl.BlockSpec((BM_T, N_NODES), lambda i: (i, 0)),
        compiler_params=tc_params,
    )(t.T)

    mesh = plsc.VectorSubcoreMesh(core_axis_name="c", subcore_axis_name="s")
    cp = pltpu.CompilerParams(needs_layout_passes=False)
    run = functools.partial(
        pl.kernel,
        mesh=mesh,
        compiler_params=cp,
        out_type=jax.ShapeDtypeStruct((BATCH,), jnp.float32),
        scratch_types=[
            pltpu.VMEM((NG, GCH), jnp.int32),
            pltpu.VMEM((2, GCH, WROW), jnp.float32),
            pltpu.VMEM((BPW, N_NODES), jnp.float32),
            pltpu.VMEM((LANES, LANES), jnp.float32),
            pltpu.VMEM((BPW,), jnp.float32),
            pltpu.SemaphoreType.DMA,
            pltpu.SemaphoreType.DMA,
        ],
    )(_sc_dot_kernel)
    return run(t2, c2, wp)
